# baseline (speedup 1.0000x reference)
import dataclasses
import functools

import jax
import jax.numpy as jnp
from jax import lax
from jax.experimental import pallas as pl
from jax.experimental.pallas import tpu as pltpu
from jax.experimental.pallas import tpu_sc as plsc

D_MODEL = 2048
D_FF = 8192
NUM_OPS = 16

BM = 1024
BF = 2048
BN = 512
NF = D_FF // BF
NN = D_MODEL // BN


def _sc_route(x16):
    mesh = plsc.VectorSubcoreMesh(core_axis_name="c", subcore_axis_name="s")

    cp = pltpu.CompilerParams()
    if "needs_layout_passes" in pltpu.CompilerParams.__dataclass_fields__:
        cp = dataclasses.replace(cp, needs_layout_passes=False)

    @functools.partial(
        pl.kernel,
        mesh=mesh,
        compiler_params=cp,
        out_type=jax.ShapeDtypeStruct((NUM_OPS,), jnp.int32),
        scratch_types=[
            pltpu.VMEM((NUM_OPS,), jnp.float32),
            pltpu.VMEM((NUM_OPS,), jnp.int32),
        ],
    )
    def route(x_hbm, op_hbm, v_ref, o_ref):
        is_leader = jnp.logical_and(
            lax.axis_index("c") == 0, lax.axis_index("s") == 0
        )

        @pl.when(is_leader)
        def _():
            pltpu.sync_copy(x_hbm, v_ref)
            v = v_ref[...]
            mx = jnp.max(v)
            idx = lax.iota(jnp.int32, NUM_OPS)
            cand = jnp.where(v == mx, idx, NUM_OPS)
            o_ref[...] = jnp.broadcast_to(jnp.min(cand), (NUM_OPS,))
            pltpu.sync_copy(o_ref, op_hbm)

    return route(x16)


def _hidden_body(op_ref, x_ref, w1_ref, b1_ref, w2c_ref, h_ref, w2b_ref):
    h = jnp.dot(x_ref[...], w1_ref[0], preferred_element_type=jnp.float32)
    h_ref[...] = jnp.maximum(h + b1_ref[0], 0.0).astype(jnp.bfloat16)
    w2b_ref[...] = w2c_ref[0].astype(jnp.bfloat16)


def _output_body(op_ref, h_ref, w2_ref, b2_ref, o_ref):
    o_ref[...] = (
        jnp.dot(h_ref[...], w2_ref[...], preferred_element_type=jnp.float32)
        + b2_ref[0]
    )


def kernel(x, W1, b1, W2, b2):
    batch, seq, d_model = x.shape
    m_total = batch * seq
    xf = x.reshape(m_total, d_model)

    op_arr = _sc_route(xf[0, :NUM_OPS])

    b1r = b1.reshape(b1.shape[0], 1, D_FF)
    b2r = b2.reshape(b2.shape[0], 1, d_model)

    n_steps = NF * (m_total // BM)
    w2_rows = D_FF // n_steps
    h, w2b = pl.pallas_call(
        _hidden_body,
        grid_spec=pltpu.PrefetchScalarGridSpec(
            num_scalar_prefetch=1,
            grid=(NF, m_total // BM),
            in_specs=[
                pl.BlockSpec((BM, d_model), lambda f, m, op: (m, 0)),
                pl.BlockSpec((1, d_model, BF), lambda f, m, op: (op[0], 0, f)),
                pl.BlockSpec((1, 1, BF), lambda f, m, op: (op[0], 0, f)),
                pl.BlockSpec(
                    (1, w2_rows, d_model),
                    lambda f, m, op: (op[0], f * (m_total // BM) + m, 0),
                ),
            ],
            out_specs=[
                pl.BlockSpec((BM, BF), lambda f, m, op: (m, f)),
                pl.BlockSpec(
                    (w2_rows, d_model),
                    lambda f, m, op: (f * (m_total // BM) + m, 0),
                ),
            ],
        ),
        out_shape=[
            jax.ShapeDtypeStruct((m_total, D_FF), jnp.bfloat16),
            jax.ShapeDtypeStruct((D_FF, d_model), jnp.bfloat16),
        ],
        compiler_params=pltpu.CompilerParams(
            dimension_semantics=("arbitrary", "arbitrary"),
            vmem_limit_bytes=64 * 1024 * 1024,
        ),
    )(op_arr, xf, W1, b1r, W2)

    out = pl.pallas_call(
        _output_body,
        grid_spec=pltpu.PrefetchScalarGridSpec(
            num_scalar_prefetch=1,
            grid=(m_total // BM, NN),
            in_specs=[
                pl.BlockSpec((BM, D_FF), lambda m, n, op: (m, 0)),
                pl.BlockSpec((D_FF, BN), lambda m, n, op: (0, n)),
                pl.BlockSpec((1, 1, BN), lambda m, n, op: (op[0], 0, n)),
            ],
            out_specs=pl.BlockSpec((BM, BN), lambda m, n, op: (m, n)),
        ),
        out_shape=jax.ShapeDtypeStruct((m_total, d_model), jnp.float32),
        compiler_params=pltpu.CompilerParams(
            dimension_semantics=("arbitrary", "arbitrary"),
            vmem_limit_bytes=64 * 1024 * 1024,
        ),
    )(op_arr, h, w2b, b2r)

    return out.reshape(batch, seq, d_model)


# speedup vs baseline: 1.0031x; 1.0031x over previous
import dataclasses
import functools

import jax
import jax.numpy as jnp
from jax import lax
from jax.experimental import pallas as pl
from jax.experimental.pallas import tpu as pltpu
from jax.experimental.pallas import tpu_sc as plsc

D_MODEL = 2048
D_FF = 8192
NUM_OPS = 16

BM = 1024
BF = 2048
BN = 512
NF = D_FF // BF
NN = D_MODEL // BN


def _sc_route(x16):
    mesh = plsc.VectorSubcoreMesh(
        core_axis_name="c", subcore_axis_name="s", num_cores=1
    )

    cp = pltpu.CompilerParams()
    if "needs_layout_passes" in pltpu.CompilerParams.__dataclass_fields__:
        cp = dataclasses.replace(cp, needs_layout_passes=False)

    @functools.partial(
        pl.kernel,
        mesh=mesh,
        compiler_params=cp,
        out_type=jax.ShapeDtypeStruct((NUM_OPS,), jnp.int32),
        scratch_types=[
            pltpu.VMEM((NUM_OPS,), jnp.float32),
            pltpu.VMEM((NUM_OPS,), jnp.int32),
        ],
    )
    def route(x_hbm, op_hbm, v_ref, o_ref):
        is_leader = jnp.logical_and(
            lax.axis_index("c") == 0, lax.axis_index("s") == 0
        )

        @pl.when(is_leader)
        def _():
            pltpu.sync_copy(x_hbm, v_ref)
            v = v_ref[...]
            mx = jnp.max(v)
            idx = lax.iota(jnp.int32, NUM_OPS)
            cand = jnp.where(v == mx, idx, NUM_OPS)
            o_ref[...] = jnp.broadcast_to(jnp.min(cand), (NUM_OPS,))
            pltpu.sync_copy(o_ref, op_hbm)

    return route(x16)


def _hidden_body(op_ref, x_ref, w1_ref, b1_ref, w2c_ref, h_ref, w2b_ref):
    h = jnp.dot(x_ref[...], w1_ref[0], preferred_element_type=jnp.float32)
    h_ref[...] = jnp.maximum(h + b1_ref[0], 0.0).astype(jnp.bfloat16)
    w2b_ref[...] = w2c_ref[0].astype(jnp.bfloat16)


def _output_body(op_ref, h_ref, w2_ref, b2_ref, o_ref):
    o_ref[...] = (
        jnp.dot(h_ref[...], w2_ref[...], preferred_element_type=jnp.float32)
        + b2_ref[0]
    )


def kernel(x, W1, b1, W2, b2):
    batch, seq, d_model = x.shape
    m_total = batch * seq
    xf = x.reshape(m_total, d_model)

    op_arr = _sc_route(xf[0, :NUM_OPS])

    b1r = b1.reshape(b1.shape[0], 1, D_FF)
    b2r = b2.reshape(b2.shape[0], 1, d_model)

    n_steps = NF * (m_total // BM)
    w2_rows = D_FF // n_steps
    h, w2b = pl.pallas_call(
        _hidden_body,
        grid_spec=pltpu.PrefetchScalarGridSpec(
            num_scalar_prefetch=1,
            grid=(NF, m_total // BM),
            in_specs=[
                pl.BlockSpec((BM, d_model), lambda f, m, op: (m, 0)),
                pl.BlockSpec((1, d_model, BF), lambda f, m, op: (op[0], 0, f)),
                pl.BlockSpec((1, 1, BF), lambda f, m, op: (op[0], 0, f)),
                pl.BlockSpec(
                    (1, w2_rows, d_model),
                    lambda f, m, op: (op[0], f * (m_total // BM) + m, 0),
                ),
            ],
            out_specs=[
                pl.BlockSpec((BM, BF), lambda f, m, op: (m, f)),
                pl.BlockSpec(
                    (w2_rows, d_model),
                    lambda f, m, op: (f * (m_total // BM) + m, 0),
                ),
            ],
        ),
        out_shape=[
            jax.ShapeDtypeStruct((m_total, D_FF), jnp.bfloat16),
            jax.ShapeDtypeStruct((D_FF, d_model), jnp.bfloat16),
        ],
        compiler_params=pltpu.CompilerParams(
            dimension_semantics=("arbitrary", "arbitrary"),
            vmem_limit_bytes=64 * 1024 * 1024,
        ),
    )(op_arr, xf, W1, b1r, W2)

    out = pl.pallas_call(
        _output_body,
        grid_spec=pltpu.PrefetchScalarGridSpec(
            num_scalar_prefetch=1,
            grid=(m_total // BM, NN),
            in_specs=[
                pl.BlockSpec((BM, D_FF), lambda m, n, op: (m, 0)),
                pl.BlockSpec((D_FF, BN), lambda m, n, op: (0, n)),
                pl.BlockSpec((1, 1, BN), lambda m, n, op: (op[0], 0, n)),
            ],
            out_specs=pl.BlockSpec((BM, BN), lambda m, n, op: (m, n)),
        ),
        out_shape=jax.ShapeDtypeStruct((m_total, d_model), jnp.float32),
        compiler_params=pltpu.CompilerParams(
            dimension_semantics=("arbitrary", "arbitrary"),
            vmem_limit_bytes=64 * 1024 * 1024,
        ),
    )(op_arr, h, w2b, b2r)

    return out.reshape(batch, seq, d_model)
